# baseline (speedup 1.0000x reference)
import functools

import jax
import jax.numpy as jnp
from jax import lax
from jax.experimental import pallas as pl
from jax.experimental.pallas import tpu as pltpu
from jax.experimental.pallas import tpu_sc as plsc

DIST_DIM = 64
LANES = 16
NUM_WORKERS = 32
ISPAN = 3200
NBUF = 2
UNROLL = 8


@functools.lru_cache(maxsize=None)
def _make_sc_call(n):
    per_w = n // NUM_WORKERS
    nblk = per_w // ISPAN
    assert n == per_w * NUM_WORKERS and per_w == nblk * ISPAN
    assert ISPAN % (LANES * UNROLL) == 0 and ISPAN % 128 == 0
    assert nblk % NBUF == 0 and nblk >= NBUF
    mesh = plsc.VectorSubcoreMesh(core_axis_name="c", subcore_axis_name="s")

    @functools.partial(
        pl.kernel,
        out_type=jax.ShapeDtypeStruct((DIST_DIM, n), jnp.float32),
        mesh=mesh,
        scratch_types=(
            [pltpu.VMEM((per_w,), jnp.float32),
             pltpu.VMEM((DIST_DIM,), jnp.float32)]
            + [pltpu.VMEM((8, ISPAN), jnp.float32)] * NBUF
            + [pltpu.SemaphoreType.DMA] * NBUF
        ),
    )
    def call(dist_hbm, w_hbm, out_hbm, dist_v, w_v, *rest):
        bufs = rest[:NBUF]
        sems = rest[NBUF:]
        cid = lax.axis_index("c")
        sid = lax.axis_index("s")
        wid = sid * 2 + cid
        i0 = wid * per_w

        pltpu.sync_copy(dist_hbm.at[pl.ds(i0, per_w)], dist_v)
        pltpu.sync_copy(w_hbm, w_v)

        def compute_block(b, buf, w8):
            out_v = bufs[buf]
            base = b * ISPAN

            def grp_body(g, carry):
                off0 = pl.multiple_of(base + g * (LANES * UNROLL),
                                      LANES * UNROLL)
                for u in range(UNROLL):
                    dvec = dist_v[pl.ds(off0 + u * LANES, LANES)]
                    col = g * (LANES * UNROLL) + u * LANES
                    for j in range(8):
                        out_v[j, pl.ds(col, LANES)] = dvec * w8[j]
                return carry

            lax.fori_loop(0, ISPAN // (LANES * UNROLL), grp_body, 0)

        def hbm_dst(a, b):
            off = pl.multiple_of(i0 + b * ISPAN, ISPAN)
            return out_hbm.at[pl.ds(a * 8, 8), pl.ds(off, ISPAN)]

        def start_dma(a, b, buf):
            pltpu.async_copy(bufs[buf], hbm_dst(a, b), sems[buf])

        def wait_dma(a, b, buf):
            pltpu.make_async_copy(bufs[buf], hbm_dst(a, b), sems[buf]).wait()

        w_regs = [w_v[pl.ds(q * LANES, LANES)] for q in range(DIST_DIM // LANES)]

        for a in range(DIST_DIM // 8):
            w8 = [
                w_regs[(a * 8 + j) // LANES].at[
                    jnp.full((LANES,), (a * 8 + j) % LANES, jnp.int32)
                ].get(mode="promise_in_bounds")
                for j in range(8)
            ]
            for buf in range(NBUF):
                if a > 0:
                    wait_dma(a - 1, jnp.int32(nblk - NBUF + buf), buf)
                compute_block(jnp.int32(buf), buf, w8)
                start_dma(a, jnp.int32(buf), buf)

            def outer(i, carry, *, a=a, w8=w8):
                b0 = i * NBUF
                for buf in range(NBUF):
                    b = b0 + buf
                    wait_dma(a, b - NBUF, buf)
                    compute_block(b, buf, w8)
                    start_dma(a, b, buf)
                return carry

            if nblk > NBUF:
                lax.fori_loop(1, nblk // NBUF, outer, 0)

        for buf in range(NBUF):
            wait_dma(DIST_DIM // 8 - 1, jnp.int32(nblk - NBUF + buf), buf)

    return call


def kernel(dist, embed_weight):
    n = dist.shape[0]
    w = embed_weight.reshape((DIST_DIM,))
    out_t = _make_sc_call(n)(dist, w)
    return out_t.T


# speedup vs baseline: 1.4878x; 1.4878x over previous
import functools

import jax
import jax.numpy as jnp
from jax import lax
from jax.experimental import pallas as pl
from jax.experimental.pallas import tpu as pltpu
from jax.experimental.pallas import tpu_sc as plsc

DIST_DIM = 64
LANES = 16
NUM_WORKERS = 32
ISPAN = 3200
NBUF = 2
UNROLL = 4


@functools.lru_cache(maxsize=None)
def _make_sc_call(n):
    per_w = n // NUM_WORKERS
    nblk = per_w // ISPAN
    assert n == per_w * NUM_WORKERS and per_w == nblk * ISPAN
    assert ISPAN % (LANES * UNROLL) == 0 and ISPAN % 128 == 0
    assert nblk % NBUF == 0 and nblk >= NBUF
    mesh = plsc.VectorSubcoreMesh(core_axis_name="c", subcore_axis_name="s")

    @functools.partial(
        pl.kernel,
        out_type=jax.ShapeDtypeStruct((DIST_DIM, n), jnp.float32),
        mesh=mesh,
        scratch_types=(
            [pltpu.VMEM((per_w,), jnp.float32),
             pltpu.VMEM((DIST_DIM,), jnp.float32)]
            + [pltpu.VMEM((8, ISPAN), jnp.float32)] * NBUF
            + [pltpu.SemaphoreType.DMA] * NBUF
        ),
    )
    def call(dist_hbm, w_hbm, out_hbm, dist_v, w_v, *rest):
        bufs = rest[:NBUF]
        sems = rest[NBUF:]
        cid = lax.axis_index("c")
        sid = lax.axis_index("s")
        wid = sid * 2 + cid
        i0 = wid * per_w

        pltpu.sync_copy(dist_hbm.at[pl.ds(i0, per_w)], dist_v)
        pltpu.sync_copy(w_hbm, w_v)

        def compute_block(b, buf, w8):
            out_v = bufs[buf]
            base = b * ISPAN

            def grp_body(g, carry):
                off0 = pl.multiple_of(base + g * (LANES * UNROLL),
                                      LANES * UNROLL)
                dvecs = [dist_v[pl.ds(off0 + u * LANES, LANES)]
                         for u in range(UNROLL)]
                col0 = g * (LANES * UNROLL)
                for j in range(8):
                    for u in range(UNROLL):
                        out_v[j, pl.ds(col0 + u * LANES, LANES)] = (
                            dvecs[u] * w8[j])
                return carry

            lax.fori_loop(0, ISPAN // (LANES * UNROLL), grp_body, 0)

        def hbm_dst(a, b):
            off = pl.multiple_of(i0 + b * ISPAN, ISPAN)
            return out_hbm.at[pl.ds(a * 8, 8), pl.ds(off, ISPAN)]

        def start_dma(a, b, buf):
            pltpu.async_copy(bufs[buf], hbm_dst(a, b), sems[buf])

        def wait_dma(a, b, buf):
            pltpu.make_async_copy(bufs[buf], hbm_dst(a, b), sems[buf]).wait()

        w_regs = [w_v[pl.ds(q * LANES, LANES)] for q in range(DIST_DIM // LANES)]

        for a in range(DIST_DIM // 8):
            w8 = [
                w_regs[(a * 8 + j) // LANES].at[
                    jnp.full((LANES,), (a * 8 + j) % LANES, jnp.int32)
                ].get(mode="promise_in_bounds")
                for j in range(8)
            ]
            for buf in range(NBUF):
                if a > 0:
                    wait_dma(a - 1, jnp.int32(nblk - NBUF + buf), buf)
                compute_block(jnp.int32(buf), buf, w8)
                start_dma(a, jnp.int32(buf), buf)

            def outer(i, carry, *, a=a, w8=w8):
                b0 = i * NBUF
                for buf in range(NBUF):
                    b = b0 + buf
                    wait_dma(a, b - NBUF, buf)
                    compute_block(b, buf, w8)
                    start_dma(a, b, buf)
                return carry

            if nblk > NBUF:
                lax.fori_loop(1, nblk // NBUF, outer, 0)

        for buf in range(NBUF):
            wait_dma(DIST_DIM // 8 - 1, jnp.int32(nblk - NBUF + buf), buf)

    return call


def kernel(dist, embed_weight):
    n = dist.shape[0]
    w = embed_weight.reshape((DIST_DIM,))
    out_t = _make_sc_call(n)(dist, w)
    return out_t.T


# speedup vs baseline: 1.4889x; 1.0007x over previous
import functools

import jax
import jax.numpy as jnp
from jax import lax
from jax.experimental import pallas as pl
from jax.experimental.pallas import tpu as pltpu
from jax.experimental.pallas import tpu_sc as plsc

DIST_DIM = 64
LANES = 16
NUM_WORKERS = 32
ISPAN = 3200
NBUF = 2
UNROLL = 8


@functools.lru_cache(maxsize=None)
def _make_sc_call(n):
    per_w = n // NUM_WORKERS
    nblk = per_w // ISPAN
    assert n == per_w * NUM_WORKERS and per_w == nblk * ISPAN
    assert ISPAN % (LANES * UNROLL) == 0 and ISPAN % 128 == 0
    assert nblk % NBUF == 0 and nblk >= NBUF
    mesh = plsc.VectorSubcoreMesh(core_axis_name="c", subcore_axis_name="s")

    @functools.partial(
        pl.kernel,
        out_type=jax.ShapeDtypeStruct((DIST_DIM, n), jnp.float32),
        mesh=mesh,
        scratch_types=(
            [pltpu.VMEM((per_w,), jnp.float32),
             pltpu.VMEM((DIST_DIM,), jnp.float32)]
            + [pltpu.VMEM((8, ISPAN), jnp.float32)] * NBUF
            + [pltpu.SemaphoreType.DMA] * NBUF
        ),
    )
    def call(dist_hbm, w_hbm, out_hbm, dist_v, w_v, *rest):
        bufs = rest[:NBUF]
        sems = rest[NBUF:]
        cid = lax.axis_index("c")
        sid = lax.axis_index("s")
        wid = sid * 2 + cid
        i0 = wid * per_w

        pltpu.sync_copy(dist_hbm.at[pl.ds(i0, per_w)], dist_v)
        pltpu.sync_copy(w_hbm, w_v)

        def compute_block(b, buf, w8):
            out_v = bufs[buf]
            base = b * ISPAN

            def grp_body(g, carry):
                off0 = pl.multiple_of(base + g * (LANES * UNROLL),
                                      LANES * UNROLL)
                dvecs = [dist_v[pl.ds(off0 + u * LANES, LANES)]
                         for u in range(UNROLL)]
                col0 = g * (LANES * UNROLL)
                for j in range(8):
                    for u in range(UNROLL):
                        out_v[j, pl.ds(col0 + u * LANES, LANES)] = (
                            dvecs[u] * w8[j])
                return carry

            lax.fori_loop(0, ISPAN // (LANES * UNROLL), grp_body, 0)

        def hbm_dst(a, b):
            off = pl.multiple_of(i0 + b * ISPAN, ISPAN)
            return out_hbm.at[pl.ds(a * 8, 8), pl.ds(off, ISPAN)]

        def start_dma(a, b, buf):
            pltpu.async_copy(bufs[buf], hbm_dst(a, b), sems[buf])

        def wait_dma(a, b, buf):
            pltpu.make_async_copy(bufs[buf], hbm_dst(a, b), sems[buf]).wait()

        w_regs = [w_v[pl.ds(q * LANES, LANES)] for q in range(DIST_DIM // LANES)]

        for a in range(DIST_DIM // 8):
            w8 = [
                w_regs[(a * 8 + j) // LANES].at[
                    jnp.full((LANES,), (a * 8 + j) % LANES, jnp.int32)
                ].get(mode="promise_in_bounds")
                for j in range(8)
            ]
            for buf in range(NBUF):
                if a > 0:
                    wait_dma(a - 1, jnp.int32(nblk - NBUF + buf), buf)
                compute_block(jnp.int32(buf), buf, w8)
                start_dma(a, jnp.int32(buf), buf)

            def outer(i, carry, *, a=a, w8=w8):
                b0 = i * NBUF
                for buf in range(NBUF):
                    b = b0 + buf
                    wait_dma(a, b - NBUF, buf)
                    compute_block(b, buf, w8)
                    start_dma(a, b, buf)
                return carry

            if nblk > NBUF:
                lax.fori_loop(1, nblk // NBUF, outer, 0)

        for buf in range(NBUF):
            wait_dma(DIST_DIM // 8 - 1, jnp.int32(nblk - NBUF + buf), buf)

    return call


def kernel(dist, embed_weight):
    n = dist.shape[0]
    w = embed_weight.reshape((DIST_DIM,))
    out_t = _make_sc_call(n)(dist, w)
    return out_t.T
